# baseline (speedup 1.0000x reference)
import functools

import jax
import jax.numpy as jnp
from jax import lax
from jax.experimental import pallas as pl
from jax.experimental.pallas import tpu as pltpu
from jax.experimental.pallas import tpu_sc as plsc

N = 10000
E = 160000
H = 256
Z = 256
K = 2

NP = 10240
NC = 2
NS = 16
NW = NC * NS
E_PAD = 163840
C1 = 64
C2 = 128
NCH1 = E_PAD // (C1 * NW)
NCH2 = E_PAD // (C2 * NS)
RPT = NP // NS
BLK = 1000

_mesh = plsc.VectorSubcoreMesh(core_axis_name="c", subcore_axis_name="s")


def _prep_body(zG_ref, xt_ref, Ws_ref, Wd_ref, F2w1_ref, U_ref, P_ref, Q_ref):
    zg = zG_ref[...]
    xt = xt_ref[...]
    for k in range(K):
        a = lax.dot_general(zg, Ws_ref[k], (((1,), (1,)), ((), ())),
                            preferred_element_type=jnp.float32)
        u = lax.dot_general(a, Wd_ref[k], (((1,), (0,)), ((), ())),
                            preferred_element_type=jnp.float32)
        U_ref[:, k * Z:(k + 1) * Z] = u
    p = lax.dot_general(xt, F2w1_ref[:, :H], (((1,), (1,)), ((), ())),
                        preferred_element_type=jnp.float32)
    q = lax.dot_general(xt, F2w1_ref[:, H:], (((1,), (1,)), ((), ())),
                        preferred_element_type=jnp.float32)
    for t in range(4):
        P_ref[t] = p[:, 64 * t:64 * (t + 1)]
        Q_ref[t] = q[:, 64 * t:64 * (t + 1)]


_prep = pl.pallas_call(
    _prep_body,
    grid=(N // BLK,),
    in_specs=[
        pl.BlockSpec((BLK, Z), lambda i: (i, 0)),
        pl.BlockSpec((BLK, H), lambda i: (i, 0)),
        pl.BlockSpec((K, H, Z), lambda i: (0, 0, 0)),
        pl.BlockSpec((K, H, Z), lambda i: (0, 0, 0)),
        pl.BlockSpec((H, 2 * H), lambda i: (0, 0)),
    ],
    out_specs=[
        pl.BlockSpec((BLK, K * Z), lambda i: (i, 0)),
        pl.BlockSpec((4, BLK, 64), lambda i: (0, i, 0)),
        pl.BlockSpec((4, BLK, 64), lambda i: (0, i, 0)),
    ],
    out_shape=[
        jax.ShapeDtypeStruct((N, K * Z), jnp.float32),
        jax.ShapeDtypeStruct((4, N, 64), jnp.float32),
        jax.ShapeDtypeStruct((4, N, 64), jnp.float32),
    ],
)


_P1_BUF = [
    pltpu.VMEM((C1,), jnp.int32),
    pltpu.VMEM((C1,), jnp.int32),
    pltpu.VMEM((C1, K * Z), jnp.float32),
    pltpu.VMEM((C1, Z), jnp.float32),
    pltpu.VMEM((C1, 16), jnp.float32),
    pltpu.SemaphoreType.DMA,
    pltpu.SemaphoreType.DMA,
]


@functools.partial(
    pl.kernel,
    out_type=[
        jax.ShapeDtypeStruct((E_PAD, 16), jnp.float32),
        jax.ShapeDtypeStruct((NC, N, 16), jnp.float32),
    ],
    mesh=_mesh,
    compiler_params=pltpu.CompilerParams(needs_layout_passes=False,
                                         use_tc_tiling_on_sc=False),
    scratch_types=_P1_BUF + _P1_BUF + [
        pltpu.VMEM((16, 16), jnp.float32),
        pltpu.VMEM((16, 16), jnp.float32),
        pltpu.VMEM((125, 16), jnp.float32),
        pltpu.VMEM_SHARED((N, 16), jnp.float32),
    ],
)
def _pass1(U_hbm, zG_hbm, src_hbm, dst_hbm, ex_hbm, dd_hbm, *sc):
    bufA = sc[0:7]
    bufB = sc[7:14]
    scr0, scr1, zbuf, ddacc = sc[14:18]
    c = lax.axis_index("c")
    s = lax.axis_index("s")
    w = s * NC + c
    tile_base = w * (E_PAD // NW)
    lanes = lax.broadcasted_iota(jnp.int32, (16,), 0)
    col0 = jnp.zeros((16,), jnp.int32)
    col1 = col0 + 1
    col2 = col0 + 2
    ones16 = jnp.zeros((16,), jnp.float32) + 1.0

    def zb(i, _):
        zbuf[i, pl.ds(0, 16)] = jnp.zeros((16,), jnp.float32)
        return 0

    lax.fori_loop(0, 125, zb, 0)
    for buf in (bufA, bufB):
        ddupd = buf[4]

        def zu(i, _):
            ddupd[i, pl.ds(0, 16)] = jnp.zeros((16,), jnp.float32)
            return 0

        lax.fori_loop(0, C1, zu, 0)
    for z5 in range(5):
        pltpu.sync_copy(zbuf, ddacc.at[pl.ds(s * (N // NS) + z5 * 125, 125)])
    plsc.subcore_barrier()

    def start_idx(j, buf):
        sidx, didx = buf[0], buf[1]
        semio = buf[5]
        base = tile_base + j * C1
        pltpu.async_copy(src_hbm.at[pl.ds(base, C1)], sidx, semio)
        pltpu.async_copy(dst_hbm.at[pl.ds(base, C1)], didx, semio)

    def wait_idx(buf):
        sidx, didx = buf[0], buf[1]
        semio = buf[5]
        pltpu.make_async_copy(src_hbm.at[pl.ds(0, C1)], sidx, semio).wait()
        pltpu.make_async_copy(dst_hbm.at[pl.ds(0, C1)], didx, semio).wait()

    def start_gathers(buf):
        sidx, didx, urows, zrows = buf[0], buf[1], buf[2], buf[3]
        semg = buf[6]
        for g in range(C1 // 16):
            dsl = pl.ds(g * 16, 16)
            didx[dsl] = lax.rem(didx[dsl], N)
        h1 = pltpu.async_copy(U_hbm.at[sidx], urows, semg)
        h2 = pltpu.async_copy(zG_hbm.at[didx], zrows, semg)
        return h1, h2

    def compute_and_out(j, buf):
        didx, urows, zrows, ddupd = buf[1], buf[2], buf[3], buf[4]
        base = tile_base + j * C1

        def group(g, _):
            def edge(t, _):
                e = g * 16 + t
                a0 = jnp.zeros((16,), jnp.float32)
                a1 = jnp.zeros((16,), jnp.float32)
                for v in range(Z // 16):
                    zv = zrows[e, pl.ds(16 * v, 16)]
                    a0 = a0 + urows[e, pl.ds(16 * v, 16)] * zv
                    a1 = a1 + urows[e, pl.ds(Z + 16 * v, 16)] * zv
                scr0[t, pl.ds(0, 16)] = a0
                scr1[t, pl.ds(0, 16)] = a1
                return 0

            lax.fori_loop(0, 16, edge, 0)

            def col(cc, rs):
                r0, r1 = rs
                colv = col0 + cc
                r0 = r0 + plsc.load_gather(scr0, [lanes, colv])
                r1 = r1 + plsc.load_gather(scr1, [lanes, colv])
                return (r0, r1)

            r0, r1 = lax.fori_loop(
                0, 16, col,
                (jnp.zeros((16,), jnp.float32), jnp.zeros((16,), jnp.float32)))
            e0 = jnp.exp(jnp.where(r0 > 0, r0, 0.01 * r0))
            e1 = jnp.exp(jnp.where(r1 > 0, r1, 0.01 * r1))
            real = (lanes + (base + g * 16)) < E
            e0 = jnp.where(real, e0, 0.0)
            e1 = jnp.where(real, e1, 0.0)
            grows = lanes + g * 16
            plsc.store_scatter(ddupd, [grows, col0], e0)
            plsc.store_scatter(ddupd, [grows, col1], e1)
            plsc.store_scatter(ddupd, [grows, col2],
                               jnp.where(real, ones16, 0.0))
            return 0

        lax.fori_loop(0, C1 // 16, group, 0)
        pltpu.sync_copy(ddupd, ex_hbm.at[pl.ds(base, C1)])
        pltpu.sync_copy(ddupd, ddacc.at[didx], add=True)

    start_idx(0, bufA)
    start_idx(1, bufB)

    def pair(i, _):
        wait_idx(bufA)
        hA = start_gathers(bufA)
        wait_idx(bufB)
        hB = start_gathers(bufB)
        hA[0].wait()
        hA[1].wait()
        compute_and_out(2 * i, bufA)
        start_idx(jnp.minimum(2 * i + 2, NCH1 - 1), bufA)
        hB[0].wait()
        hB[1].wait()
        compute_and_out(2 * i + 1, bufB)
        start_idx(jnp.minimum(2 * i + 3, NCH1 - 1), bufB)
        return 0

    lax.fori_loop(0, NCH1 // 2, pair, 0)
    wait_idx(bufA)
    wait_idx(bufB)
    plsc.subcore_barrier()
    rsl = pl.ds(s * (N // NS), N // NS)
    pltpu.sync_copy(ddacc.at[rsl], dd_hbm.at[c, rsl])


_P2_BUF = [
    pltpu.VMEM((C2,), jnp.int32),
    pltpu.VMEM((C2,), jnp.int32),
    pltpu.VMEM((C2, 64), jnp.float32),
    pltpu.VMEM((C2, 64), jnp.float32),
    pltpu.VMEM((C2, 64), jnp.float32),
    pltpu.VMEM((C2, 16), jnp.float32),
    pltpu.SemaphoreType.DMA,
    pltpu.SemaphoreType.DMA,
    pltpu.SemaphoreType.DMA,
]


@functools.partial(
    pl.kernel,
    out_type=jax.ShapeDtypeStruct((NC, 4, N, 64), jnp.float32),
    mesh=_mesh,
    compiler_params=pltpu.CompilerParams(needs_layout_passes=False,
                                         use_tc_tiling_on_sc=False),
    scratch_types=_P2_BUF + _P2_BUF + [
        pltpu.VMEM((125, 64), jnp.float32),
        pltpu.VMEM_SHARED((N, 64), jnp.float32),
    ],
)
def _pass2(P_hbm, Q_hbm, src_hbm, dst_hbm, ex_hbm, S_hbm, *sc):
    bufA = sc[0:9]
    bufB = sc[9:18]
    zbuf, acc = sc[18:20]
    c = lax.axis_index("c")
    s = lax.axis_index("s")
    tile_base = s * (E_PAD // NS)
    is_k0 = c == 0

    def zb(i, _):
        zv = jnp.zeros((16,), jnp.float32)
        for t in range(4):
            zbuf[i, pl.ds(16 * t, 16)] = zv
        return 0

    lax.fori_loop(0, 125, zb, 0)

    def start_idx(j, buf):
        sidx, didx, exb = buf[0], buf[1], buf[5]
        semio = buf[6]
        base = tile_base + j * C2
        pltpu.async_copy(src_hbm.at[pl.ds(base, C2)], sidx, semio)
        pltpu.async_copy(dst_hbm.at[pl.ds(base, C2)], didx, semio)
        pltpu.async_copy(ex_hbm.at[pl.ds(base, C2)], exb, semio)

    def wait_idx(buf):
        sidx, didx, exb = buf[0], buf[1], buf[5]
        semio = buf[6]
        pltpu.make_async_copy(src_hbm.at[pl.ds(0, C2)], sidx, semio).wait()
        pltpu.make_async_copy(dst_hbm.at[pl.ds(0, C2)], didx, semio).wait()
        pltpu.make_async_copy(ex_hbm.at[pl.ds(0, C2)], exb, semio).wait()

    def start_gathers(q, buf):
        sidx, didx, prows, qrows = buf[0], buf[1], buf[2], buf[3]
        semg = buf[7]
        for g in range(C2 // 16):
            dsl = pl.ds(g * 16, 16)
            didx[dsl] = lax.rem(didx[dsl], N)
        h1 = pltpu.async_copy(P_hbm.at[q].at[sidx], prows, semg)
        h2 = pltpu.async_copy(Q_hbm.at[q].at[didx], qrows, semg)
        return h1, h2

    def compute_and_out(buf):
        didx, prows, qrows, upd, exb = (
            buf[1], buf[2], buf[3], buf[4], buf[5])
        semsc = buf[8]

        def edge(e, _):
            xv = exb[e, pl.ds(0, 16)]
            x = jnp.where(is_k0, xv[0], xv[1])
            for tt in range(4):
                wv = jnp.maximum(
                    prows[e, pl.ds(16 * tt, 16)]
                    + qrows[e, pl.ds(16 * tt, 16)], 0.0)
                upd[e, pl.ds(16 * tt, 16)] = wv * x
            return 0

        lax.fori_loop(0, C2, edge, 0)
        return pltpu.async_copy(upd, acc.at[didx], semsc, add=True)

    def slice_pass(q, _):
        for z5 in range(5):
            pltpu.sync_copy(zbuf, acc.at[pl.ds(s * (N // NS) + z5 * 125, 125)])
        plsc.subcore_barrier()
        start_idx(0, bufA)
        start_idx(1, bufB)

        def pair(i, _):
            wait_idx(bufA)
            hA = start_gathers(q, bufA)
            wait_idx(bufB)
            hB = start_gathers(q, bufB)
            hA[0].wait()
            hA[1].wait()
            hsA = compute_and_out(bufA)
            hB[0].wait()
            hB[1].wait()
            hsB = compute_and_out(bufB)
            hsA.wait()
            start_idx(jnp.minimum(2 * i + 2, NCH2 - 1), bufA)
            hsB.wait()
            start_idx(jnp.minimum(2 * i + 3, NCH2 - 1), bufB)
            return 0

        lax.fori_loop(0, NCH2 // 2, pair, 0)
        wait_idx(bufA)
        wait_idx(bufB)
        plsc.subcore_barrier()
        rsl = pl.ds(s * (N // NS), N // NS)
        pltpu.sync_copy(acc.at[rsl], S_hbm.at[c, q, rsl])
        plsc.subcore_barrier()
        return 0

    lax.fori_loop(0, 4, slice_pass, 0)


def _post_body(S_ref, dd_ref, F1w1_ref, F1w2_ref, F2w2_ref, out_ref):
    dds = dd_ref[0] + dd_ref[1]
    deg = jnp.maximum(dds[:, 2:3], 1.0)
    d0 = dds[:, 0:1] * deg
    d1 = dds[:, 1:2] * deg
    d0 = jnp.where(d0 > 0, d0, 1.0)
    d1 = jnp.where(d1 > 0, d1, 1.0)
    s0 = jnp.concatenate([S_ref[0, t] for t in range(4)], axis=1)
    s1 = jnp.concatenate([S_ref[1, t] for t in range(4)], axis=1)
    t0 = s0 / d0
    t1 = s1 / d1
    c0 = lax.dot_general(F1w1_ref[:, :H], F2w2_ref[...], (((1,), (0,)), ((), ())),
                         preferred_element_type=jnp.float32)
    c1 = lax.dot_general(F1w1_ref[:, H:], F2w2_ref[...], (((1,), (0,)), ((), ())),
                         preferred_element_type=jnp.float32)
    o = lax.dot_general(t0, c0, (((1,), (1,)), ((), ())),
                        preferred_element_type=jnp.float32)
    o = o + lax.dot_general(t1, c1, (((1,), (1,)), ((), ())),
                            preferred_element_type=jnp.float32)
    o = jnp.maximum(o, 0.0)
    out_ref[...] = lax.dot_general(o, F1w2_ref[...], (((1,), (1,)), ((), ())),
                                   preferred_element_type=jnp.float32)


_post = pl.pallas_call(
    _post_body,
    grid=(N // BLK,),
    in_specs=[
        pl.BlockSpec((NC, 4, BLK, 64), lambda i: (0, 0, i, 0)),
        pl.BlockSpec((NC, BLK, 16), lambda i: (0, i, 0)),
        pl.BlockSpec((H, 2 * H), lambda i: (0, 0)),
        pl.BlockSpec((H, H), lambda i: (0, 0)),
        pl.BlockSpec((H, H), lambda i: (0, 0)),
    ],
    out_specs=pl.BlockSpec((BLK, H), lambda i: (i, 0)),
    out_shape=jax.ShapeDtypeStruct((N, H), jnp.float32),
)


def kernel(zG, xt_enc, edge_index, Ws, Wd, F1_w1, F1_w2, F2_w1, F2_w2):
    ei = edge_index.astype(jnp.int32)
    pj = jnp.arange(E_PAD - E, dtype=jnp.int32)
    src = jnp.concatenate([ei[0], pj % N])
    dst = jnp.concatenate([ei[1], N + pj % (NP - N)])
    U_tab, P_tab, Q_tab = _prep(zG, xt_enc, Ws, Wd, F2_w1)
    ex, dd = _pass1(U_tab, zG, src, dst)
    S = _pass2(P_tab, Q_tab, src, dst, ex)
    return _post(S, dd, F1_w1, F1_w2, F2_w2)


# speedup vs baseline: 1.0084x; 1.0084x over previous
import functools

import jax
import jax.numpy as jnp
from jax import lax
from jax.experimental import pallas as pl
from jax.experimental.pallas import tpu as pltpu
from jax.experimental.pallas import tpu_sc as plsc

N = 10000
E = 160000
H = 256
Z = 256
K = 2

NP = 10240
NC = 2
NS = 16
NW = NC * NS
E_PAD = 163840
C1 = 64
C2 = 128
NCH1 = E_PAD // (C1 * NW)
NCH2 = E_PAD // (C2 * NS)
RPT = NP // NS
BLK = 1000

_mesh = plsc.VectorSubcoreMesh(core_axis_name="c", subcore_axis_name="s")


def _prep_body(zG_ref, xt_ref, Ws_ref, Wd_ref, F2w1_ref, U_ref, P_ref, Q_ref):
    zg = zG_ref[...]
    xt = xt_ref[...]
    for k in range(K):
        a = lax.dot_general(zg, Ws_ref[k], (((1,), (1,)), ((), ())),
                            preferred_element_type=jnp.float32)
        u = lax.dot_general(a, Wd_ref[k], (((1,), (0,)), ((), ())),
                            preferred_element_type=jnp.float32)
        U_ref[:, k * Z:(k + 1) * Z] = u
    p = lax.dot_general(xt, F2w1_ref[:, :H], (((1,), (1,)), ((), ())),
                        preferred_element_type=jnp.float32)
    q = lax.dot_general(xt, F2w1_ref[:, H:], (((1,), (1,)), ((), ())),
                        preferred_element_type=jnp.float32)
    for t in range(4):
        P_ref[t] = p[:, 64 * t:64 * (t + 1)]
        Q_ref[t] = q[:, 64 * t:64 * (t + 1)]


_prep = pl.pallas_call(
    _prep_body,
    grid=(N // BLK,),
    in_specs=[
        pl.BlockSpec((BLK, Z), lambda i: (i, 0)),
        pl.BlockSpec((BLK, H), lambda i: (i, 0)),
        pl.BlockSpec((K, H, Z), lambda i: (0, 0, 0)),
        pl.BlockSpec((K, H, Z), lambda i: (0, 0, 0)),
        pl.BlockSpec((H, 2 * H), lambda i: (0, 0)),
    ],
    out_specs=[
        pl.BlockSpec((BLK, K * Z), lambda i: (i, 0)),
        pl.BlockSpec((4, BLK, 64), lambda i: (0, i, 0)),
        pl.BlockSpec((4, BLK, 64), lambda i: (0, i, 0)),
    ],
    out_shape=[
        jax.ShapeDtypeStruct((N, K * Z), jnp.float32),
        jax.ShapeDtypeStruct((4, N, 64), jnp.float32),
        jax.ShapeDtypeStruct((4, N, 64), jnp.float32),
    ],
)


_P1_BUF = [
    pltpu.VMEM((C1,), jnp.int32),
    pltpu.VMEM((C1,), jnp.int32),
    pltpu.VMEM((C1, K * Z), jnp.float32),
    pltpu.VMEM((C1, Z), jnp.float32),
    pltpu.VMEM((C1, 16), jnp.float32),
    pltpu.SemaphoreType.DMA,
    pltpu.SemaphoreType.DMA,
]


@functools.partial(
    pl.kernel,
    out_type=[
        jax.ShapeDtypeStruct((E_PAD, 16), jnp.float32),
        jax.ShapeDtypeStruct((NC, N, 16), jnp.float32),
    ],
    mesh=_mesh,
    compiler_params=pltpu.CompilerParams(needs_layout_passes=False,
                                         use_tc_tiling_on_sc=False),
    scratch_types=_P1_BUF + _P1_BUF + [
        pltpu.VMEM((16, 16), jnp.float32),
        pltpu.VMEM((16, 16), jnp.float32),
        pltpu.VMEM((125, 16), jnp.float32),
        pltpu.VMEM_SHARED((N, 16), jnp.float32),
    ],
)
def _pass1(U_hbm, zG_hbm, src_hbm, dst_hbm, ex_hbm, dd_hbm, *sc):
    bufA = sc[0:7]
    bufB = sc[7:14]
    scr0, scr1, zbuf, ddacc = sc[14:18]
    c = lax.axis_index("c")
    s = lax.axis_index("s")
    w = s * NC + c
    tile_base = w * (E_PAD // NW)
    lanes = lax.broadcasted_iota(jnp.int32, (16,), 0)
    col0 = jnp.zeros((16,), jnp.int32)
    col1 = col0 + 1
    col2 = col0 + 2
    ones16 = jnp.zeros((16,), jnp.float32) + 1.0

    def zb(i, _):
        zbuf[i, pl.ds(0, 16)] = jnp.zeros((16,), jnp.float32)
        return 0

    lax.fori_loop(0, 125, zb, 0)
    for buf in (bufA, bufB):
        ddupd = buf[4]

        def zu(i, _):
            ddupd[i, pl.ds(0, 16)] = jnp.zeros((16,), jnp.float32)
            return 0

        lax.fori_loop(0, C1, zu, 0)
    for z5 in range(5):
        pltpu.sync_copy(zbuf, ddacc.at[pl.ds(s * (N // NS) + z5 * 125, 125)])
    plsc.subcore_barrier()

    def start_idx(j, buf):
        sidx, didx = buf[0], buf[1]
        semio = buf[5]
        base = tile_base + j * C1
        pltpu.async_copy(src_hbm.at[pl.ds(base, C1)], sidx, semio)
        pltpu.async_copy(dst_hbm.at[pl.ds(base, C1)], didx, semio)

    def wait_idx(buf):
        sidx, didx = buf[0], buf[1]
        semio = buf[5]
        pltpu.make_async_copy(src_hbm.at[pl.ds(0, C1)], sidx, semio).wait()
        pltpu.make_async_copy(dst_hbm.at[pl.ds(0, C1)], didx, semio).wait()

    def start_gathers(buf):
        sidx, didx, urows, zrows = buf[0], buf[1], buf[2], buf[3]
        semg = buf[6]
        for g in range(C1 // 16):
            dsl = pl.ds(g * 16, 16)
            didx[dsl] = lax.rem(didx[dsl], N)
        h1 = pltpu.async_copy(U_hbm.at[sidx], urows, semg)
        h2 = pltpu.async_copy(zG_hbm.at[didx], zrows, semg)
        return h1, h2

    def compute_and_out(j, buf):
        didx, urows, zrows, ddupd = buf[1], buf[2], buf[3], buf[4]
        base = tile_base + j * C1

        def group(g, _):
            def edge(t, _):
                e = g * 16 + t
                a0 = jnp.zeros((16,), jnp.float32)
                a1 = jnp.zeros((16,), jnp.float32)
                for v in range(Z // 16):
                    zv = zrows[e, pl.ds(16 * v, 16)]
                    a0 = a0 + urows[e, pl.ds(16 * v, 16)] * zv
                    a1 = a1 + urows[e, pl.ds(Z + 16 * v, 16)] * zv
                scr0[t, pl.ds(0, 16)] = a0
                scr1[t, pl.ds(0, 16)] = a1
                return 0

            lax.fori_loop(0, 16, edge, 0)

            def col(cc, rs):
                r0, r1 = rs
                colv = col0 + cc
                r0 = r0 + plsc.load_gather(scr0, [lanes, colv])
                r1 = r1 + plsc.load_gather(scr1, [lanes, colv])
                return (r0, r1)

            r0, r1 = lax.fori_loop(
                0, 16, col,
                (jnp.zeros((16,), jnp.float32), jnp.zeros((16,), jnp.float32)))
            e0 = jnp.exp(jnp.where(r0 > 0, r0, 0.01 * r0))
            e1 = jnp.exp(jnp.where(r1 > 0, r1, 0.01 * r1))
            real = (lanes + (base + g * 16)) < E
            e0 = jnp.where(real, e0, 0.0)
            e1 = jnp.where(real, e1, 0.0)
            grows = lanes + g * 16
            plsc.store_scatter(ddupd, [grows, col0], e0)
            plsc.store_scatter(ddupd, [grows, col1], e1)
            plsc.store_scatter(ddupd, [grows, col2],
                               jnp.where(real, ones16, 0.0))
            return 0

        lax.fori_loop(0, C1 // 16, group, 0)
        pltpu.sync_copy(ddupd, ex_hbm.at[pl.ds(base, C1)])
        pltpu.sync_copy(ddupd, ddacc.at[didx], add=True)

    start_idx(0, bufA)
    start_idx(1, bufB)

    def pair(i, _):
        wait_idx(bufA)
        hA = start_gathers(bufA)
        wait_idx(bufB)
        hB = start_gathers(bufB)
        hA[0].wait()
        hA[1].wait()
        compute_and_out(2 * i, bufA)
        start_idx(jnp.minimum(2 * i + 2, NCH1 - 1), bufA)
        hB[0].wait()
        hB[1].wait()
        compute_and_out(2 * i + 1, bufB)
        start_idx(jnp.minimum(2 * i + 3, NCH1 - 1), bufB)
        return 0

    lax.fori_loop(0, NCH1 // 2, pair, 0)
    wait_idx(bufA)
    wait_idx(bufB)
    plsc.subcore_barrier()
    rsl = pl.ds(s * (N // NS), N // NS)
    pltpu.sync_copy(ddacc.at[rsl], dd_hbm.at[c, rsl])


_P2_BUF = [
    pltpu.VMEM((C2,), jnp.int32),
    pltpu.VMEM((C2,), jnp.int32),
    pltpu.VMEM((C2, 64), jnp.float32),
    pltpu.VMEM((C2, 64), jnp.float32),
    pltpu.VMEM((C2, 64), jnp.float32),
    pltpu.VMEM((C2, 16), jnp.float32),
    pltpu.SemaphoreType.DMA,
    pltpu.SemaphoreType.DMA,
]


@functools.partial(
    pl.kernel,
    out_type=jax.ShapeDtypeStruct((NC, 4, N, 64), jnp.float32),
    mesh=_mesh,
    compiler_params=pltpu.CompilerParams(needs_layout_passes=False,
                                         use_tc_tiling_on_sc=False),
    scratch_types=_P2_BUF + _P2_BUF + [
        pltpu.VMEM((125, 64), jnp.float32),
        pltpu.VMEM_SHARED((N, 64), jnp.float32),
    ],
)
def _pass2(P_hbm, Q_hbm, src_hbm, dst_hbm, ex_hbm, S_hbm, *sc):
    bufA = sc[0:8]
    bufB = sc[8:16]
    zbuf, acc = sc[16:18]
    c = lax.axis_index("c")
    s = lax.axis_index("s")
    tile_base = s * (E_PAD // NS)
    is_k0 = c == 0

    def zb(i, _):
        zv = jnp.zeros((16,), jnp.float32)
        for t in range(4):
            zbuf[i, pl.ds(16 * t, 16)] = zv
        return 0

    lax.fori_loop(0, 125, zb, 0)

    def start_idx(j, buf):
        sidx, didx, exb = buf[0], buf[1], buf[5]
        semio = buf[6]
        base = tile_base + j * C2
        pltpu.async_copy(src_hbm.at[pl.ds(base, C2)], sidx, semio)
        pltpu.async_copy(dst_hbm.at[pl.ds(base, C2)], didx, semio)
        pltpu.async_copy(ex_hbm.at[pl.ds(base, C2)], exb, semio)

    def wait_idx(buf):
        sidx, didx, exb = buf[0], buf[1], buf[5]
        semio = buf[6]
        pltpu.make_async_copy(src_hbm.at[pl.ds(0, C2)], sidx, semio).wait()
        pltpu.make_async_copy(dst_hbm.at[pl.ds(0, C2)], didx, semio).wait()
        pltpu.make_async_copy(ex_hbm.at[pl.ds(0, C2)], exb, semio).wait()

    def start_gathers(q, buf):
        sidx, didx, prows, qrows = buf[0], buf[1], buf[2], buf[3]
        semg = buf[7]
        for g in range(C2 // 16):
            dsl = pl.ds(g * 16, 16)
            didx[dsl] = lax.rem(didx[dsl], N)
        h1 = pltpu.async_copy(P_hbm.at[q].at[sidx], prows, semg)
        h2 = pltpu.async_copy(Q_hbm.at[q].at[didx], qrows, semg)
        return h1, h2

    def compute_and_out(buf):
        didx, prows, qrows, upd, exb = (
            buf[1], buf[2], buf[3], buf[4], buf[5])

        def edge(e, _):
            xv = exb[e, pl.ds(0, 16)]
            x = jnp.where(is_k0, xv[0], xv[1])
            for tt in range(4):
                wv = jnp.maximum(
                    prows[e, pl.ds(16 * tt, 16)]
                    + qrows[e, pl.ds(16 * tt, 16)], 0.0)
                upd[e, pl.ds(16 * tt, 16)] = wv * x
            return 0

        lax.fori_loop(0, C2, edge, 0)
        pltpu.sync_copy(upd, acc.at[didx], add=True)

    def slice_pass(q, _):
        for z5 in range(5):
            pltpu.sync_copy(zbuf, acc.at[pl.ds(s * (N // NS) + z5 * 125, 125)])
        plsc.subcore_barrier()
        start_idx(0, bufA)
        start_idx(1, bufB)

        def pair(i, _):
            wait_idx(bufA)
            hA = start_gathers(q, bufA)
            wait_idx(bufB)
            hB = start_gathers(q, bufB)
            hA[0].wait()
            hA[1].wait()
            compute_and_out(bufA)
            start_idx(jnp.minimum(2 * i + 2, NCH2 - 1), bufA)
            hB[0].wait()
            hB[1].wait()
            compute_and_out(bufB)
            start_idx(jnp.minimum(2 * i + 3, NCH2 - 1), bufB)
            return 0

        lax.fori_loop(0, NCH2 // 2, pair, 0)
        wait_idx(bufA)
        wait_idx(bufB)
        plsc.subcore_barrier()
        rsl = pl.ds(s * (N // NS), N // NS)
        pltpu.sync_copy(acc.at[rsl], S_hbm.at[c, q, rsl])
        plsc.subcore_barrier()
        return 0

    lax.fori_loop(0, 4, slice_pass, 0)


def _post_body(S_ref, dd_ref, F1w1_ref, F1w2_ref, F2w2_ref, out_ref):
    dds = dd_ref[0] + dd_ref[1]
    deg = jnp.maximum(dds[:, 2:3], 1.0)
    d0 = dds[:, 0:1] * deg
    d1 = dds[:, 1:2] * deg
    d0 = jnp.where(d0 > 0, d0, 1.0)
    d1 = jnp.where(d1 > 0, d1, 1.0)
    s0 = jnp.concatenate([S_ref[0, t] for t in range(4)], axis=1)
    s1 = jnp.concatenate([S_ref[1, t] for t in range(4)], axis=1)
    t0 = s0 / d0
    t1 = s1 / d1
    c0 = lax.dot_general(F1w1_ref[:, :H], F2w2_ref[...], (((1,), (0,)), ((), ())),
                         preferred_element_type=jnp.float32)
    c1 = lax.dot_general(F1w1_ref[:, H:], F2w2_ref[...], (((1,), (0,)), ((), ())),
                         preferred_element_type=jnp.float32)
    o = lax.dot_general(t0, c0, (((1,), (1,)), ((), ())),
                        preferred_element_type=jnp.float32)
    o = o + lax.dot_general(t1, c1, (((1,), (1,)), ((), ())),
                            preferred_element_type=jnp.float32)
    o = jnp.maximum(o, 0.0)
    out_ref[...] = lax.dot_general(o, F1w2_ref[...], (((1,), (1,)), ((), ())),
                                   preferred_element_type=jnp.float32)


_post = pl.pallas_call(
    _post_body,
    grid=(N // BLK,),
    in_specs=[
        pl.BlockSpec((NC, 4, BLK, 64), lambda i: (0, 0, i, 0)),
        pl.BlockSpec((NC, BLK, 16), lambda i: (0, i, 0)),
        pl.BlockSpec((H, 2 * H), lambda i: (0, 0)),
        pl.BlockSpec((H, H), lambda i: (0, 0)),
        pl.BlockSpec((H, H), lambda i: (0, 0)),
    ],
    out_specs=pl.BlockSpec((BLK, H), lambda i: (i, 0)),
    out_shape=jax.ShapeDtypeStruct((N, H), jnp.float32),
)


def kernel(zG, xt_enc, edge_index, Ws, Wd, F1_w1, F1_w2, F2_w1, F2_w2):
    ei = edge_index.astype(jnp.int32)
    pj = jnp.arange(E_PAD - E, dtype=jnp.int32)
    src = jnp.concatenate([ei[0], pj % N])
    dst = jnp.concatenate([ei[1], N + pj % (NP - N)])
    U_tab, P_tab, Q_tab = _prep(zG, xt_enc, Ws, Wd, F2_w1)
    ex, dd = _pass1(U_tab, zG, src, dst)
    S = _pass2(P_tab, Q_tab, src, dst, ex)
    return _post(S, dd, F1_w1, F1_w2, F2_w2)
